# baseline (speedup 1.0000x reference)
import jax
import jax.numpy as jnp
from jax import lax
from jax.experimental import pallas as pl
from jax.experimental.pallas import tpu as pltpu
from jax.experimental.pallas import tpu_sc as plsc

N = 10000
E = 320000
D = 128
DP = D // 2
NC = 2
NS = 16
NW = NC * NS
ET = E // NW
K = 80
NCH = ET // K
ZB = 200
ZNB = N // ZB
ZK = (ZNB + NS - 1) // NS

_MESH = plsc.VectorSubcoreMesh(
    core_axis_name="c", subcore_axis_name="s", num_cores=NC, num_subcores=NS)
_ILV = plsc.PackFormat.INTERLEAVED
_CPARAMS = pltpu.CompilerParams(needs_layout_passes=False,
                               use_tc_tiling_on_sc=False)


def _scale_rows(gb, vb, sb):
    def body(i16, carry):
        vvec = vb[pl.ds(i16 * 16, 16)]
        for e in range(16):
            i = i16 * 16 + e
            v0 = vvec[e]
            for g in range(4):
                w = gb[i, pl.ds(g * 16, 16)]
                a = plsc.bitcast(w << 16, jnp.float32)
                b = plsc.bitcast(w & jnp.int32(-65536), jnp.float32)
                sb[i, pl.ds(g * 32, 16)] = a * v0
                sb[i, pl.ds(g * 32 + 16, 16)] = b * v0
        return carry
    lax.fori_loop(0, K // 16, body, 0)


def _spmm_body(x_hbm, rows_hbm, cols_hbm, vals_hbm, z_hbm, part_hbm,
               acc, cb0, cb1, vb0, vb1, rb0, rb1, rb2, rb3,
               gb0, gb1, sb0, sb1,
               sz, sg0, sg1, sc0, sc1, sv0, sv1,
               sr0, sr1, sr2, sr3, ss0, ss1):
    c = lax.axis_index("c")
    s = lax.axis_index("s")
    wid = c * NS + s
    base = wid * ET

    cbs, vbs = (cb0, cb1), (vb0, vb1)
    rbs = (rb0, rb1, rb2, rb3)
    gbs, sbs = (gb0, gb1), (sb0, sb1)
    sgs, scs, svs = (sg0, sg1), (sc0, sc1), (sv0, sv1)
    srs = (sr0, sr1, sr2, sr3)
    sss = (ss0, ss1)

    def off_of(g):
        return pl.multiple_of(base + g * K, 8)

    def fetch_cols(g, s2):
        pltpu.async_copy(cols_hbm.at[pl.ds(off_of(g), K)], cbs[s2], scs[s2])

    def fetch_vals(g, s2):
        pltpu.async_copy(vals_hbm.at[pl.ds(off_of(g), K)], vbs[s2], svs[s2])

    def fetch_rows(g, s4):
        pltpu.async_copy(rows_hbm.at[pl.ds(off_of(g), K)], rbs[s4], srs[s4])

    def wait_cols(s2):
        pltpu.make_async_copy(cols_hbm.at[pl.ds(0, K)], cbs[s2], scs[s2]).wait()

    def wait_vals(s2):
        pltpu.make_async_copy(vals_hbm.at[pl.ds(0, K)], vbs[s2], svs[s2]).wait()

    def wait_rows(s4):
        pltpu.make_async_copy(rows_hbm.at[pl.ds(0, K)], rbs[s4], srs[s4]).wait()

    def issue_gather(s2):
        pltpu.async_copy(x_hbm.at[cbs[s2]], gbs[s2], sgs[s2])

    def wait_gather(s2):
        pltpu.make_async_copy(x_hbm.at[cbs[s2]], gbs[s2], sgs[s2]).wait()

    def issue_scatter(s2, s4):
        pltpu.async_copy(sbs[s2], acc.at[rbs[s4]], sss[s2], add=True)

    def wait_scatter(s2, s4):
        pltpu.make_async_copy(sbs[s2], acc.at[rbs[s4]], sss[s2]).wait()

    for g in (0, 1):
        fetch_cols(g, g)
        fetch_vals(g, g)
        fetch_rows(g, g)
    for g in (0, 1):
        wait_cols(g)
        issue_gather(g)

    for k in range(ZK):
        b = k * NS + s

        @pl.when(b < ZNB)
        def _():
            off = pl.multiple_of(b * ZB, 8)
            pltpu.async_copy(z_hbm, acc.at[pl.ds(off, ZB)], sz)
    for k in range(ZK):
        b = k * NS + s

        @pl.when(b < ZNB)
        def _():
            pltpu.make_async_copy(z_hbm, acc.at[pl.ds(0, ZB)], sz).wait()
    plsc.subcore_barrier()

    def section(G, s2, s4, p, first, fetch_next):
        wait_gather(s2)

        if fetch_next:
            fetch_cols(G + 2, s2)
        elif fetch_next is None:
            @pl.when(G + 2 < NCH)
            def _():
                fetch_cols(G + 2, s2)

        if first:
            @pl.when(p > 0)
            def _():
                wait_scatter(s2, s4)
        else:
            wait_scatter(s2, s4)

        if fetch_next:
            fetch_rows(G + 2, (s4 + 2) % 4)
        elif fetch_next is None:
            @pl.when(G + 2 < NCH)
            def _():
                fetch_rows(G + 2, (s4 + 2) % 4)

        wait_vals(s2)
        _scale_rows(gbs[s2], vbs[s2], sbs[s2])

        if fetch_next:
            fetch_vals(G + 2, s2)
        elif fetch_next is None:
            @pl.when(G + 2 < NCH)
            def _():
                fetch_vals(G + 2, s2)

        wait_rows(s4)
        issue_scatter(s2, s4)

        if fetch_next:
            wait_cols(s2)
            issue_gather(s2)
        elif fetch_next is None:
            @pl.when(G + 2 < NCH)
            def _():
                wait_cols(s2)
                issue_gather(s2)

    NQ = NCH // 4

    def quad_body(p, carry):
        g0 = 4 * p
        section(g0 + 0, 0, 0, p, True, True)
        section(g0 + 1, 1, 1, p, True, True)
        section(g0 + 2, 0, 2, p, False, None)
        section(g0 + 3, 1, 3, p, False, None)
        return carry

    lax.fori_loop(0, NQ, quad_body, 0)

    if NCH % 4 == 1:
        wait_gather(0)
        wait_scatter(0, 0)
        wait_vals(0)
        _scale_rows(gbs[0], vbs[0], sbs[0])
        wait_rows(0)
        issue_scatter(0, 0)
        wait_scatter(1, 3)
        wait_scatter(0, 0)
    plsc.subcore_barrier()

    for k in range(ZK):
        b = k * NS + s

        @pl.when(b < ZNB)
        def _():
            off = pl.multiple_of(b * ZB, 8)
            dst = pl.multiple_of(c * N + b * ZB, 8)
            pltpu.async_copy(acc.at[pl.ds(off, ZB)],
                             part_hbm.at[pl.ds(dst, ZB)], sz)
    for k in range(ZK):
        b = k * NS + s

        @pl.when(b < ZNB)
        def _():
            pltpu.make_async_copy(acc.at[pl.ds(0, ZB)],
                                  part_hbm.at[pl.ds(0, ZB)], sz).wait()


def _spmm(xp, rows, cols, vals, zrows):
    return pl.kernel(
        _spmm_body,
        out_type=jax.ShapeDtypeStruct((NC * N, D), jnp.float32),
        mesh=_MESH,
        compiler_params=_CPARAMS,
        scratch_types=[
            pltpu.VMEM_SHARED((N, D), jnp.float32),
            pltpu.VMEM((K,), jnp.int32), pltpu.VMEM((K,), jnp.int32),
            pltpu.VMEM((K,), jnp.float32), pltpu.VMEM((K,), jnp.float32),
            pltpu.VMEM((K,), jnp.int32), pltpu.VMEM((K,), jnp.int32),
            pltpu.VMEM((K,), jnp.int32), pltpu.VMEM((K,), jnp.int32),
            pltpu.VMEM((K, DP), jnp.int32), pltpu.VMEM((K, DP), jnp.int32),
            pltpu.VMEM((K, D), jnp.float32), pltpu.VMEM((K, D), jnp.float32),
        ] + [pltpu.SemaphoreType.DMA] * 13,
    )(xp, rows, cols, vals, zrows)


BLK = 80
NBLK = N // BLK
KMAX = (NBLK + NW - 1) // NW


def _make_combine_body(weights, scale, f32_out, pack_out):
    n_in = len(weights)
    n_out = int(f32_out) + int(pack_out)

    def body(*refs):
        in_hbms = refs[:n_in]
        outs = refs[n_in:n_in + n_out]
        sc = refs[n_in + n_out:]
        ibufs = (sc[:n_in], sc[n_in:2 * n_in])
        sc = sc[2 * n_in:]
        if f32_out:
            out_f32 = outs[0]
            of32 = (sc[0], sc[1])
            sc = sc[2:]
        if pack_out:
            out_i32 = outs[-1]
            oi32 = (sc[0], sc[1])
            sc = sc[2:]
        sin = (sc[0], sc[1])
        sout = (sc[2], sc[3])
        c = lax.axis_index("c")
        s = lax.axis_index("s")
        wid = c * NS + s

        def issue_in(b, sl):
            r0 = pl.multiple_of(b * BLK, 8)
            for t in range(n_in):
                pltpu.async_copy(in_hbms[t].at[pl.ds(r0, BLK)],
                                 ibufs[sl][t], sin[sl])

        def wait_in(sl):
            for t in range(n_in):
                pltpu.make_async_copy(in_hbms[t].at[pl.ds(0, BLK)],
                                      ibufs[sl][t], sin[sl]).wait()

        def compute(sl):
            bufs = ibufs[sl]

            def rbody(i, carry):
                vs = []
                for j in range(D // 16):
                    slc = pl.ds(j * 16, 16)
                    v = bufs[0][i, slc] * (weights[0] * scale)
                    for t in range(1, n_in):
                        v = v + bufs[t][i, slc] * (weights[t] * scale)
                    if f32_out:
                        of32[sl][i, slc] = v
                    vs.append(v)
                if pack_out:
                    for g in range(4):
                        pk = plsc.pack(vs[2 * g], vs[2 * g + 1], format=_ILV)
                        oi32[sl][i, pl.ds(g * 16, 16)] = plsc.bitcast(
                            pk, jnp.int32)
                return carry
            lax.fori_loop(0, BLK, rbody, 0)

        def issue_out(b, sl):
            r0 = pl.multiple_of(b * BLK, 8)
            if f32_out:
                pltpu.async_copy(of32[sl], out_f32.at[pl.ds(r0, BLK)],
                                 sout[sl])
            if pack_out:
                pltpu.async_copy(oi32[sl], out_i32.at[pl.ds(r0, BLK)],
                                 sout[sl])

        def wait_out(sl):
            if f32_out:
                pltpu.make_async_copy(of32[sl], out_f32.at[pl.ds(0, BLK)],
                                      sout[sl]).wait()
            if pack_out:
                pltpu.make_async_copy(oi32[sl], out_i32.at[pl.ds(0, BLK)],
                                      sout[sl]).wait()

        issue_in(wid, 0)
        for k in range(KMAX):
            b = k * NW + wid
            nxt = b + NW
            sl = k % 2
            if k + 1 < KMAX:
                if (k + 1) * NW + NW - 1 < NBLK:
                    issue_in(nxt, (k + 1) % 2)
                else:
                    @pl.when(nxt < NBLK)
                    def _():
                        issue_in(nxt, (k + 1) % 2)
            if k * NW + NW - 1 < NBLK:
                wait_in(sl)
                if k >= 2:
                    wait_out(sl)
                compute(sl)
                issue_out(b, sl)
            else:
                @pl.when(b < NBLK)
                def _():
                    wait_in(sl)
                    if k >= 2:
                        wait_out(sl)
                    compute(sl)
                    issue_out(b, sl)
        wait_out(0)
        wait_out(1)
    return body


def _combine(arrs, weights, scale=1.0, f32_out=True, pack_out=False):
    body = _make_combine_body(tuple(weights), scale, f32_out, pack_out)
    n_in = len(arrs)
    out_type = []
    if f32_out:
        out_type.append(jax.ShapeDtypeStruct((N, D), jnp.float32))
    if pack_out:
        out_type.append(jax.ShapeDtypeStruct((N, DP), jnp.int32))
    scratch = [pltpu.VMEM((BLK, D), jnp.float32) for _ in range(2 * n_in)]
    if f32_out:
        scratch += [pltpu.VMEM((BLK, D), jnp.float32)] * 2
    if pack_out:
        scratch += [pltpu.VMEM((BLK, DP), jnp.int32)] * 2
    scratch += [pltpu.SemaphoreType.DMA] * 4
    res = pl.kernel(
        body,
        out_type=tuple(out_type),
        mesh=_MESH,
        compiler_params=_CPARAMS,
        scratch_types=scratch,
    )(*arrs)
    return res


def _pack_x(x):
    xg = x.reshape(N, 4, 2, 16)
    inter = jnp.stack([xg[:, :, 0, :], xg[:, :, 1, :]], axis=-1)
    xb = inter.reshape(N, D).astype(jnp.bfloat16)
    return jax.lax.bitcast_convert_type(xb.reshape(N, DP, 2), jnp.int32)


def kernel(pois_embs, hg_up_vals, hg_pu_vals, hg_up_index, hg_pu_index):
    up_rows = hg_up_index[0]
    up_cols = hg_up_index[1]
    pu_rows = hg_pu_index[0]
    pu_cols = hg_pu_index[1]
    zrows = jnp.zeros((ZB, D), jnp.float32)

    x0 = pois_embs
    xp0 = _pack_x(x0)
    p = _spmm(xp0, up_rows, up_cols, hg_up_vals, zrows)
    (msg1p,) = _combine([p[:N], p[N:]], [1.0, 1.0],
                        f32_out=False, pack_out=True)
    p = _spmm(msg1p, pu_rows, pu_cols, hg_pu_vals, zrows)
    x1, x1p = _combine([p[:N], p[N:], x0], [1.0, 1.0, 1.0], pack_out=True)
    p = _spmm(x1p, up_rows, up_cols, hg_up_vals, zrows)
    (msg2p,) = _combine([p[:N], p[N:]], [1.0, 1.0],
                        f32_out=False, pack_out=True)
    p = _spmm(msg2p, pu_rows, pu_cols, hg_pu_vals, zrows)
    (out,) = _combine([p[:N], p[N:], x0, x1], [1.0, 1.0, 1.0, 2.0],
                      scale=1.0 / 3.0)
    return out


# speedup vs baseline: 1.3671x; 1.3671x over previous
import jax
import jax.numpy as jnp
from jax import lax
from jax.experimental import pallas as pl
from jax.experimental.pallas import tpu as pltpu
from jax.experimental.pallas import tpu_sc as plsc

N = 10000
E = 320000
D = 128
NC = 2
NS = 16
NW = NC * NS
ET = E // NW
K = 80
NCH = ET // K
ZB = 200
ZNB = N // ZB
ZK = (ZNB + NS - 1) // NS

_MESH = plsc.VectorSubcoreMesh(
    core_axis_name="c", subcore_axis_name="s", num_cores=NC, num_subcores=NS)


def _scale_rows(gb, vb, sb):
    def body(i16, carry):
        vvec = vb[pl.ds(i16 * 16, 16)]
        for e in range(16):
            i = i16 * 16 + e
            v0 = vvec[e]
            for j in range(D // 16):
                sl = pl.ds(j * 16, 16)
                sb[i, sl] = gb[i, sl] * v0
        return carry
    lax.fori_loop(0, K // 16, body, 0)


def _spmm_body(x_hbm, rows_hbm, cols_hbm, vals_hbm, z_hbm, part_hbm,
               acc, cb0, cb1, vb0, vb1, rb0, rb1, rb2, rb3,
               gb0, gb1, sb0, sb1,
               sz, sg0, sg1, sc0, sc1, sv0, sv1,
               sr0, sr1, sr2, sr3, ss0, ss1):
    c = lax.axis_index("c")
    s = lax.axis_index("s")
    wid = c * NS + s
    base = wid * ET

    cbs, vbs = (cb0, cb1), (vb0, vb1)
    rbs = (rb0, rb1, rb2, rb3)
    gbs, sbs = (gb0, gb1), (sb0, sb1)
    sgs, scs, svs = (sg0, sg1), (sc0, sc1), (sv0, sv1)
    srs = (sr0, sr1, sr2, sr3)
    sss = (ss0, ss1)

    def off_of(g):
        return pl.multiple_of(base + g * K, 8)

    def fetch_cols(g, s2):
        pltpu.async_copy(cols_hbm.at[pl.ds(off_of(g), K)], cbs[s2], scs[s2])

    def fetch_vals(g, s2):
        pltpu.async_copy(vals_hbm.at[pl.ds(off_of(g), K)], vbs[s2], svs[s2])

    def fetch_rows(g, s4):
        pltpu.async_copy(rows_hbm.at[pl.ds(off_of(g), K)], rbs[s4], srs[s4])

    def wait_cols(s2):
        pltpu.make_async_copy(cols_hbm.at[pl.ds(0, K)], cbs[s2], scs[s2]).wait()

    def wait_vals(s2):
        pltpu.make_async_copy(vals_hbm.at[pl.ds(0, K)], vbs[s2], svs[s2]).wait()

    def wait_rows(s4):
        pltpu.make_async_copy(rows_hbm.at[pl.ds(0, K)], rbs[s4], srs[s4]).wait()

    def issue_gather(s2):
        pltpu.async_copy(x_hbm.at[cbs[s2]], gbs[s2], sgs[s2])

    def wait_gather(s2):
        pltpu.make_async_copy(x_hbm.at[cbs[s2]], gbs[s2], sgs[s2]).wait()

    def issue_scatter(s2, s4):
        pltpu.async_copy(sbs[s2], acc.at[rbs[s4]], sss[s2], add=True)

    def wait_scatter(s2, s4):
        pltpu.make_async_copy(sbs[s2], acc.at[rbs[s4]], sss[s2]).wait()

    for g in (0, 1):
        fetch_cols(g, g)
        fetch_vals(g, g)
        fetch_rows(g, g)
    for g in (0, 1):
        wait_cols(g)
        issue_gather(g)

    for k in range(ZK):
        b = k * NS + s

        @pl.when(b < ZNB)
        def _():
            off = pl.multiple_of(b * ZB, 8)
            pltpu.async_copy(z_hbm, acc.at[pl.ds(off, ZB)], sz)
    for k in range(ZK):
        b = k * NS + s

        @pl.when(b < ZNB)
        def _():
            pltpu.make_async_copy(z_hbm, acc.at[pl.ds(0, ZB)], sz).wait()
    plsc.subcore_barrier()

    def section(G, s2, s4, p, first, fetch_next):
        wait_gather(s2)

        if fetch_next:
            fetch_cols(G + 2, s2)
        elif fetch_next is None:
            @pl.when(G + 2 < NCH)
            def _():
                fetch_cols(G + 2, s2)

        if first:
            @pl.when(p > 0)
            def _():
                wait_scatter(s2, s4)
        else:
            wait_scatter(s2, s4)

        if fetch_next:
            fetch_rows(G + 2, (s4 + 2) % 4)
        elif fetch_next is None:
            @pl.when(G + 2 < NCH)
            def _():
                fetch_rows(G + 2, (s4 + 2) % 4)

        wait_vals(s2)
        _scale_rows(gbs[s2], vbs[s2], sbs[s2])

        if fetch_next:
            fetch_vals(G + 2, s2)
        elif fetch_next is None:
            @pl.when(G + 2 < NCH)
            def _():
                fetch_vals(G + 2, s2)

        wait_rows(s4)
        issue_scatter(s2, s4)

        if fetch_next:
            wait_cols(s2)
            issue_gather(s2)
        elif fetch_next is None:
            @pl.when(G + 2 < NCH)
            def _():
                wait_cols(s2)
                issue_gather(s2)

    NQ = NCH // 4

    def quad_body(p, carry):
        g0 = 4 * p
        section(g0 + 0, 0, 0, p, True, True)
        section(g0 + 1, 1, 1, p, True, True)
        section(g0 + 2, 0, 2, p, False, None)
        section(g0 + 3, 1, 3, p, False, None)
        return carry

    lax.fori_loop(0, NQ, quad_body, 0)

    if NCH % 4 == 1:
        wait_gather(0)
        wait_scatter(0, 0)
        wait_vals(0)
        _scale_rows(gbs[0], vbs[0], sbs[0])
        wait_rows(0)
        issue_scatter(0, 0)
        wait_scatter(1, 3)
        wait_scatter(0, 0)
    plsc.subcore_barrier()

    for k in range(ZK):
        b = k * NS + s

        @pl.when(b < ZNB)
        def _():
            off = pl.multiple_of(b * ZB, 8)
            dst = pl.multiple_of(c * N + b * ZB, 8)
            pltpu.async_copy(acc.at[pl.ds(off, ZB)],
                             part_hbm.at[pl.ds(dst, ZB)], sz)
    for k in range(ZK):
        b = k * NS + s

        @pl.when(b < ZNB)
        def _():
            pltpu.make_async_copy(acc.at[pl.ds(0, ZB)],
                                  part_hbm.at[pl.ds(0, ZB)], sz).wait()


def _spmm(xp, rows, cols, vals, zrows):
    return pl.kernel(
        _spmm_body,
        out_type=jax.ShapeDtypeStruct((NC * N, D), jnp.float32),
        mesh=_MESH,
        scratch_types=[
            pltpu.VMEM_SHARED((N, D), jnp.float32),
            pltpu.VMEM((K,), jnp.int32), pltpu.VMEM((K,), jnp.int32),
            pltpu.VMEM((K,), jnp.float32), pltpu.VMEM((K,), jnp.float32),
            pltpu.VMEM((K,), jnp.int32), pltpu.VMEM((K,), jnp.int32),
            pltpu.VMEM((K,), jnp.int32), pltpu.VMEM((K,), jnp.int32),
            pltpu.VMEM((K, D), jnp.float32), pltpu.VMEM((K, D), jnp.float32),
            pltpu.VMEM((K, D), jnp.float32), pltpu.VMEM((K, D), jnp.float32),
        ] + [pltpu.SemaphoreType.DMA] * 13,
    )(xp, rows, cols, vals, zrows)


BLK = 80
NBLK = N // BLK
KMAX = (NBLK + NW - 1) // NW


def _make_combine_body(weights, scale):
    n_in = len(weights)

    def body(*refs):
        in_hbms = refs[:n_in]
        out_f32 = refs[n_in]
        sc = refs[n_in + 1:]
        ibufs = (sc[:n_in], sc[n_in:2 * n_in])
        sc = sc[2 * n_in:]
        of32 = (sc[0], sc[1])
        sin = (sc[2], sc[3])
        sout = (sc[4], sc[5])
        c = lax.axis_index("c")
        s = lax.axis_index("s")
        wid = c * NS + s

        def issue_in(b, sl):
            r0 = pl.multiple_of(b * BLK, 8)
            for t in range(n_in):
                pltpu.async_copy(in_hbms[t].at[pl.ds(r0, BLK)],
                                 ibufs[sl][t], sin[sl])

        def wait_in(sl):
            for t in range(n_in):
                pltpu.make_async_copy(in_hbms[t].at[pl.ds(0, BLK)],
                                      ibufs[sl][t], sin[sl]).wait()

        def compute(sl):
            bufs = ibufs[sl]

            def rbody(i, carry):
                for j in range(D // 16):
                    slc = pl.ds(j * 16, 16)
                    v = bufs[0][i, slc] * (weights[0] * scale)
                    for t in range(1, n_in):
                        v = v + bufs[t][i, slc] * (weights[t] * scale)
                    of32[sl][i, slc] = v
                return carry
            lax.fori_loop(0, BLK, rbody, 0)

        def issue_out(b, sl):
            r0 = pl.multiple_of(b * BLK, 8)
            pltpu.async_copy(of32[sl], out_f32.at[pl.ds(r0, BLK)], sout[sl])

        def wait_out(sl):
            pltpu.make_async_copy(of32[sl], out_f32.at[pl.ds(0, BLK)],
                                  sout[sl]).wait()

        issue_in(wid, 0)
        for k in range(KMAX):
            b = k * NW + wid
            nxt = b + NW
            sl = k % 2
            if k + 1 < KMAX:
                if (k + 1) * NW + NW - 1 < NBLK:
                    issue_in(nxt, (k + 1) % 2)
                else:
                    @pl.when(nxt < NBLK)
                    def _():
                        issue_in(nxt, (k + 1) % 2)
            if k * NW + NW - 1 < NBLK:
                wait_in(sl)
                if k >= 2:
                    wait_out(sl)
                compute(sl)
                issue_out(b, sl)
            else:
                @pl.when(b < NBLK)
                def _():
                    wait_in(sl)
                    if k >= 2:
                        wait_out(sl)
                    compute(sl)
                    issue_out(b, sl)
        wait_out(0)
        wait_out(1)
    return body


def _combine(arrs, weights, scale=1.0):
    body = _make_combine_body(tuple(weights), scale)
    n_in = len(arrs)
    scratch = [pltpu.VMEM((BLK, D), jnp.float32) for _ in range(2 * n_in + 2)]
    scratch += [pltpu.SemaphoreType.DMA] * 4
    return pl.kernel(
        body,
        out_type=jax.ShapeDtypeStruct((N, D), jnp.float32),
        mesh=_MESH,
        scratch_types=scratch,
    )(*arrs)


def kernel(pois_embs, hg_up_vals, hg_pu_vals, hg_up_index, hg_pu_index):
    up_rows = hg_up_index[0]
    up_cols = hg_up_index[1]
    pu_rows = hg_pu_index[0]
    pu_cols = hg_pu_index[1]
    zrows = jnp.zeros((ZB, D), jnp.float32)

    x0 = pois_embs
    p = _spmm(x0, up_rows, up_cols, hg_up_vals, zrows)
    msg1 = _combine([p[:N], p[N:]], [1.0, 1.0])
    p = _spmm(msg1, pu_rows, pu_cols, hg_pu_vals, zrows)
    x1 = _combine([p[:N], p[N:], x0], [1.0, 1.0, 1.0])
    p = _spmm(x1, up_rows, up_cols, hg_up_vals, zrows)
    msg2 = _combine([p[:N], p[N:]], [1.0, 1.0])
    p = _spmm(msg2, pu_rows, pu_cols, hg_pu_vals, zrows)
    return _combine([p[:N], p[N:], x0, x1], [1.0, 1.0, 1.0, 2.0],
                    scale=1.0 / 3.0)
